# baseline (speedup 1.0000x reference)
import functools

import jax
import jax.numpy as jnp
from jax import lax
from jax.experimental import pallas as pl
from jax.experimental.pallas import tpu as pltpu
from jax.experimental.pallas import tpu_sc as plsc

NUM_CORES = 2
NUM_SUBCORES = 16
NUM_WORKERS = NUM_CORES * NUM_SUBCORES

NBUF = 2
PAD_D = 128


def _gather_rows(batch: int, hist: int, dim: int):
  b_per_w = batch // NUM_WORKERS
  mesh = plsc.VectorSubcoreMesh(
      core_axis_name="c", subcore_axis_name="s",
      num_cores=NUM_CORES, num_subcores=NUM_SUBCORES)

  @functools.partial(
      pl.kernel,
      mesh=mesh,
      out_type=jax.ShapeDtypeStruct((batch, hist, dim), jnp.float32),
      scratch_types=[
          pltpu.VMEM((b_per_w, hist), jnp.int32),
          pltpu.VMEM((NBUF, hist, PAD_D), jnp.float32),
          pltpu.SemaphoreType.DMA((NBUF,)),
          pltpu.SemaphoreType.DMA((NBUF,)),
      ],
      compiler_params=pltpu.CompilerParams(use_tc_tiling_on_sc=False),
  )
  def grab(idx_hbm, table_hbm, out_hbm, idx_v, rows_v, gsem, wsem):
    wid = lax.axis_index("s") * NUM_CORES + lax.axis_index("c")
    base = wid * b_per_w
    pltpu.sync_copy(idx_hbm.at[pl.ds(base, b_per_w)], idx_v)

    for b in range(NBUF):
      pltpu.async_copy(
          table_hbm.at[idx_v.at[b]], rows_v.at[b], gsem.at[b])

    @pl.loop(0, b_per_w, step=NBUF)
    def _step(i):
      for b in range(NBUF):
        cur = i + b
        pltpu.make_async_copy(
            table_hbm.at[idx_v.at[cur]], rows_v.at[b], gsem.at[b]).wait()
        pltpu.async_copy(
            rows_v.at[b, :, pl.ds(0, dim)], out_hbm.at[base + cur],
            wsem.at[b])
        nxt = cur + NBUF

        @pl.when(nxt < b_per_w)
        def _():
          pltpu.make_async_copy(
              rows_v.at[b, :, pl.ds(0, dim)], out_hbm.at[base + cur],
              wsem.at[b]).wait()
          pltpu.async_copy(
              table_hbm.at[idx_v.at[nxt]], rows_v.at[b], gsem.at[b])

    for b in range(NBUF):
      cur = b_per_w - NBUF + b
      pltpu.make_async_copy(
          rows_v.at[b, :, pl.ds(0, dim)], out_hbm.at[base + cur],
          wsem.at[b]).wait()

  return grab


def kernel(items, weight):
  batch, hist = items.shape
  vocab, dim = weight.shape
  wpad = jnp.pad(weight, ((0, 0), (0, PAD_D - dim)))
  return _gather_rows(batch, hist, dim)(items.astype(jnp.int32), wpad)


# speedup vs baseline: 1.2352x; 1.2352x over previous
import functools

import jax
import jax.numpy as jnp
from jax import lax
from jax.experimental import pallas as pl
from jax.experimental.pallas import tpu as pltpu
from jax.experimental.pallas import tpu_sc as plsc

NUM_CORES = 2
NUM_SUBCORES = 16
NUM_WORKERS = NUM_CORES * NUM_SUBCORES

NBUF = 2
PAD_D = 128


def _gather_rows(batch: int, hist: int, dim: int):
  b_per_w = batch // NUM_WORKERS
  mesh = plsc.VectorSubcoreMesh(
      core_axis_name="c", subcore_axis_name="s",
      num_cores=NUM_CORES, num_subcores=NUM_SUBCORES)

  @functools.partial(
      pl.kernel,
      mesh=mesh,
      out_type=jax.ShapeDtypeStruct((batch, hist, PAD_D), jnp.float32),
      scratch_types=[
          pltpu.VMEM((b_per_w, hist), jnp.int32),
          pltpu.VMEM((NBUF, hist, PAD_D), jnp.float32),
          pltpu.SemaphoreType.DMA((NBUF,)),
          pltpu.SemaphoreType.DMA((NBUF,)),
      ],
      compiler_params=pltpu.CompilerParams(use_tc_tiling_on_sc=False),
  )
  def grab(idx_hbm, table_hbm, out_hbm, idx_v, rows_v, gsem, wsem):
    wid = lax.axis_index("s") * NUM_CORES + lax.axis_index("c")
    base = wid * b_per_w
    pltpu.sync_copy(idx_hbm.at[pl.ds(base, b_per_w)], idx_v)

    for b in range(NBUF):
      pltpu.async_copy(
          table_hbm.at[idx_v.at[b]], rows_v.at[b], gsem.at[b])

    @pl.loop(0, b_per_w, step=NBUF)
    def _step(i):
      for b in range(NBUF):
        cur = i + b
        pltpu.make_async_copy(
            table_hbm.at[idx_v.at[cur]], rows_v.at[b], gsem.at[b]).wait()
        pltpu.async_copy(
            rows_v.at[b], out_hbm.at[base + cur], wsem.at[b])
        nxt = cur + NBUF

        @pl.when(nxt < b_per_w)
        def _():
          pltpu.make_async_copy(
              rows_v.at[b], out_hbm.at[base + cur], wsem.at[b]).wait()
          pltpu.async_copy(
              table_hbm.at[idx_v.at[nxt]], rows_v.at[b], gsem.at[b])

    for b in range(NBUF):
      cur = b_per_w - NBUF + b
      pltpu.make_async_copy(
          rows_v.at[b], out_hbm.at[base + cur], wsem.at[b]).wait()

  return grab


def kernel(items, weight):
  batch, hist = items.shape
  vocab, dim = weight.shape
  wpad = jnp.pad(weight, ((0, 0), (0, PAD_D - dim)))
  out = _gather_rows(batch, hist, dim)(items.astype(jnp.int32), wpad)
  return out[:, :, :dim]


# speedup vs baseline: 1.4447x; 1.1696x over previous
import functools

import jax
import jax.numpy as jnp
from jax import lax
from jax.experimental import pallas as pl
from jax.experimental.pallas import tpu as pltpu
from jax.experimental.pallas import tpu_sc as plsc

NUM_CORES = 2
NUM_SUBCORES = 16
NUM_WORKERS = NUM_CORES * NUM_SUBCORES

NBUF = 2
PAD_D = 128


def _gather_rows(batch: int, hist: int, dim: int, vocab: int):
  b_per_w = batch // NUM_WORKERS
  mesh = plsc.VectorSubcoreMesh(
      core_axis_name="c", subcore_axis_name="s",
      num_cores=NUM_CORES, num_subcores=NUM_SUBCORES)

  @functools.partial(
      pl.kernel,
      mesh=mesh,
      out_type=jax.ShapeDtypeStruct((batch, hist, PAD_D), jnp.float32),
      scratch_types=[
          pltpu.VMEM((b_per_w, hist), jnp.int32),
          pltpu.VMEM((NBUF, hist, dim), jnp.float32),
          pltpu.SemaphoreType.DMA((NBUF,)),
          pltpu.SemaphoreType.DMA((NBUF,)),
      ],
      compiler_params=pltpu.CompilerParams(use_tc_tiling_on_sc=False),
  )
  def grab(idx_hbm, table_hbm, out_hbm, idx_v, rows_v, gsem, wsem):
    wid = lax.axis_index("s") * NUM_CORES + lax.axis_index("c")
    base = wid * b_per_w
    pltpu.sync_copy(idx_hbm.at[pl.ds(base, b_per_w)], idx_v)

    for b in range(NBUF):
      pltpu.async_copy(
          table_hbm.at[idx_v.at[b]], rows_v.at[b], gsem.at[b])

    @pl.loop(0, b_per_w, step=NBUF)
    def _step(i):
      for b in range(NBUF):
        cur = i + b
        pltpu.make_async_copy(
            table_hbm.at[idx_v.at[cur]], rows_v.at[b], gsem.at[b]).wait()
        pltpu.async_copy(
            rows_v.at[b], out_hbm.at[base + cur, :, pl.ds(0, dim)],
            wsem.at[b])
        nxt = cur + NBUF

        @pl.when(nxt < b_per_w)
        def _():
          pltpu.make_async_copy(
              rows_v.at[b], out_hbm.at[base + cur, :, pl.ds(0, dim)],
              wsem.at[b]).wait()
          pltpu.async_copy(
              table_hbm.at[idx_v.at[nxt]], rows_v.at[b], gsem.at[b])

    for b in range(NBUF):
      cur = b_per_w - NBUF + b
      pltpu.make_async_copy(
          rows_v.at[b], out_hbm.at[base + cur, :, pl.ds(0, dim)],
          wsem.at[b]).wait()

  return grab


def kernel(items, weight):
  batch, hist = items.shape
  vocab, dim = weight.shape
  wpad = jnp.pad(weight, ((0, 0), (0, PAD_D - dim)))
  w2 = wpad.reshape(vocab * (PAD_D // dim), dim)
  items2 = items.astype(jnp.int32) * (PAD_D // dim)
  out = _gather_rows(batch, hist, dim, vocab)(items2, w2)
  return out[:, :, :dim]


# speedup vs baseline: 1.4475x; 1.0019x over previous
import functools

import jax
import jax.numpy as jnp
from jax import lax
from jax.experimental import pallas as pl
from jax.experimental.pallas import tpu as pltpu
from jax.experimental.pallas import tpu_sc as plsc

NUM_CORES = 2
NUM_SUBCORES = 16
NUM_WORKERS = NUM_CORES * NUM_SUBCORES

NBUF = 4
PAD_D = 128


def _gather_rows(batch: int, hist: int, dim: int):
  b_per_w = batch // NUM_WORKERS
  assert b_per_w % NBUF == 0
  mesh = plsc.VectorSubcoreMesh(
      core_axis_name="c", subcore_axis_name="s",
      num_cores=NUM_CORES, num_subcores=NUM_SUBCORES)

  @functools.partial(
      pl.kernel,
      mesh=mesh,
      out_type=jax.ShapeDtypeStruct((batch, hist, PAD_D), jnp.float32),
      scratch_types=[
          pltpu.VMEM((b_per_w, hist), jnp.int32),
          pltpu.VMEM((NBUF, hist, dim), jnp.float32),
          pltpu.SemaphoreType.DMA((NBUF,)),
          pltpu.SemaphoreType.DMA((NBUF,)),
      ],
      compiler_params=pltpu.CompilerParams(use_tc_tiling_on_sc=False),
  )
  def grab(idx_hbm, table_hbm, out_hbm, idx_v, rows_v, gsem, wsem):
    wid = lax.axis_index("s") * NUM_CORES + lax.axis_index("c")
    base = wid * b_per_w
    pltpu.sync_copy(idx_hbm.at[pl.ds(base, b_per_w)], idx_v)

    for b in range(NBUF):
      pltpu.async_copy(
          table_hbm.at[idx_v.at[b]], rows_v.at[b], gsem.at[b])

    @pl.loop(0, b_per_w, step=NBUF)
    def _step(i):
      for b in range(NBUF):
        cur = i + b
        pltpu.make_async_copy(
            table_hbm.at[idx_v.at[cur]], rows_v.at[b], gsem.at[b]).wait()
        pltpu.async_copy(
            rows_v.at[b], out_hbm.at[base + cur, :, pl.ds(0, dim)],
            wsem.at[b])
        nxt = cur + NBUF

        @pl.when(nxt < b_per_w)
        def _():
          pltpu.make_async_copy(
              rows_v.at[b], out_hbm.at[base + cur, :, pl.ds(0, dim)],
              wsem.at[b]).wait()
          pltpu.async_copy(
              table_hbm.at[idx_v.at[nxt]], rows_v.at[b], gsem.at[b])

    for b in range(NBUF):
      cur = b_per_w - NBUF + b
      pltpu.make_async_copy(
          rows_v.at[b], out_hbm.at[base + cur, :, pl.ds(0, dim)],
          wsem.at[b]).wait()

  return grab


def kernel(items, weight):
  batch, hist = items.shape
  vocab, dim = weight.shape
  wpad = jnp.pad(weight, ((0, 0), (0, PAD_D - dim)))
  w2 = wpad.reshape(vocab * (PAD_D // dim), dim)
  items2 = items.astype(jnp.int32) * (PAD_D // dim)
  out = _gather_rows(batch, hist, dim)(items2, w2)
  return out[:, :, :dim]
